# baseline (speedup 1.0000x reference)
import functools

import jax
import jax.numpy as jnp
from jax import lax
from jax.experimental import pallas as pl
from jax.experimental.pallas import tpu as pltpu
from jax.experimental.pallas import tpu_sc as plsc

NC = 2
NS = 16
NW = NC * NS
CH = 96
D = 256
DH = 128


def _make_sc_agg(n_chunks: int, spmem_rows: int, stripe: int, tab_rows: int):
  mesh = plsc.VectorSubcoreMesh(core_axis_name="c", subcore_axis_name="s")

  @functools.partial(
      pl.kernel,
      out_type=(
          jax.ShapeDtypeStruct((NC, spmem_rows, DH), jnp.float32),
          jax.ShapeDtypeStruct((NW, spmem_rows), jnp.float32),
      ),
      mesh=mesh,
      compiler_params=pltpu.CompilerParams(needs_layout_passes=False),
      scratch_types=[
          pltpu.VMEM((n_chunks, CH), jnp.int32),
          pltpu.VMEM((n_chunks, CH), jnp.int32),
          pltpu.VMEM((CH, DH), jnp.float32),
          pltpu.VMEM((CH, DH), jnp.float32),
          pltpu.VMEM((CH, DH), jnp.float32),
          pltpu.VMEM((CH, DH), jnp.float32),
          pltpu.VMEM((spmem_rows,), jnp.float32),
          pltpu.VMEM_SHARED((spmem_rows, DH), jnp.float32),
          [pltpu.SemaphoreType.DMA] * 4,
          [pltpu.SemaphoreType.DMA] * 2,
      ],
  )
  def sc_agg(tab_hbm, src_hbm, dst_hbm, zd_hbm, zw_hbm,
             agg_hbm, deg_hbm,
             src_v, dst_v, buf0, buf1, buf2, buf3, deg_v, acc_sh,
             gsems, ssems):
    cid = lax.axis_index("c")
    sid = lax.axis_index("s")
    wid = sid * NC + cid
    tab = tab_hbm.at[pl.ds(0, tab_rows), pl.ds(cid * DH, DH)]
    cp1 = pltpu.async_copy(src_hbm.at[sid], src_v, ssems[1])
    cp2 = pltpu.async_copy(dst_hbm.at[sid], dst_v, ssems[1])
    cp3 = pltpu.async_copy(zw_hbm, deg_v, ssems[1])
    cp4 = pltpu.async_copy(zd_hbm, acc_sh.at[pl.ds(sid * stripe, stripe)],
                           ssems[1])
    cp1.wait()
    cp2.wait()
    cp3.wait()
    cp4.wait()
    plsc.subcore_barrier()

    bufs = (buf0, buf1, buf2, buf3)
    ones16 = jnp.ones((16,), jnp.float32)

    for t in range(3):
      pltpu.async_copy(tab.at[src_v.at[t]], bufs[t], gsems[t])

    def body(k, _):
      for b in range(4):
        j = k * 4 + b
        t = j + 3
        bt = (b + 3) % 4

        @pl.when(j < n_chunks)
        def _():
          pltpu.make_async_copy(tab.at[src_v.at[j]], bufs[b],
                                gsems[b]).wait()

          @pl.when(j >= 1)
          def _():
            pltpu.make_async_copy(bufs[(b + 3) % 4],
                                  acc_sh.at[dst_v.at[j]], ssems[0]).wait()

          pltpu.async_copy(bufs[b], acc_sh.at[dst_v.at[j]], ssems[0],
                           add=True)

          @pl.when(t < n_chunks)
          def _():
            pltpu.async_copy(tab.at[src_v.at[t]], bufs[bt], gsems[bt])

          @pl.when(cid == b % 2)
          def _():
            for g in range(CH // 16):
              v = dst_v[j, pl.ds(g * 16, 16)]
              plsc.addupdate_scatter(deg_v, [v], ones16)

      return 0

    lax.fori_loop(0, (n_chunks + 3) // 4, body, 0)
    pltpu.make_async_copy(bufs[0], acc_sh.at[dst_v.at[0]],
                          ssems[0]).wait()
    ecp = pltpu.async_copy(deg_v, deg_hbm.at[wid], ssems[1])
    plsc.subcore_barrier()
    pltpu.sync_copy(acc_sh.at[pl.ds(sid * stripe, stripe)],
                    agg_hbm.at[cid, pl.ds(sid * stripe, stripe)])
    ecp.wait()

  return sc_agg


def _tc_body(x_ref, agg_ref, deg_ref, ws_ref, wn_ref, b_ref, o_ref, *,
             relu):
  xs = x_ref[...]
  a = jnp.concatenate([agg_ref[0], agg_ref[1]], axis=1)
  deg = jnp.maximum(deg_ref[...], 1.0)
  h_neigh = a / deg
  dn = (((1,), (1,)), ((), ()))
  acc = lax.dot_general(xs, ws_ref[...], dn,
                        preferred_element_type=jnp.float32)
  acc = acc + lax.dot_general(h_neigh, wn_ref[...], dn,
                              preferred_element_type=jnp.float32)
  acc = acc + b_ref[...]
  if relu:
    acc = jnp.maximum(acc, 0.0)
  o_ref[...] = acc


def _tc_layer(x, agg, deg, w_self, w_neigh, b, n_out, blk, relu):
  grid = (n_out // blk,)
  return pl.pallas_call(
      functools.partial(_tc_body, relu=relu),
      grid=grid,
      in_specs=[
          pl.BlockSpec((blk, D), lambda i: (i, 0)),
          pl.BlockSpec((NC, blk, DH), lambda i: (0, i, 0)),
          pl.BlockSpec((blk, 1), lambda i: (i, 0)),
          pl.BlockSpec((D, D), lambda i: (0, 0)),
          pl.BlockSpec((D, D), lambda i: (0, 0)),
          pl.BlockSpec((1, D), lambda i: (0, 0)),
      ],
      out_specs=pl.BlockSpec((blk, D), lambda i: (i, 0)),
      out_shape=jax.ShapeDtypeStruct((n_out, D), jnp.float32),
  )(x, agg, deg, w_self, w_neigh, b.reshape(1, D))


N_DST1, N_DST2 = 5000, 2500
E1, E2 = 160000, 80000
E1_PAD = 161280
E2_PAD = 81408
N1_CHUNKS = E1_PAD // (NS * CH)
N2_CHUNKS = E2_PAD // (NS * CH)
ROWS1, STRIPE1 = 5120, 320
ROWS2, STRIPE2 = 2560, 160

_sc_agg1 = _make_sc_agg(N1_CHUNKS, ROWS1, STRIPE1, N_DST1)
_sc_agg2 = _make_sc_agg(N2_CHUNKS, ROWS2, STRIPE2, N_DST1)


def _pad_edges(ei, e_pad, n_chunks, pad_dst):
  pad = e_pad - ei.shape[1]
  src = jnp.concatenate([ei[0], jnp.zeros((pad,), ei.dtype)])
  dst = jnp.concatenate([ei[1], jnp.full((pad,), pad_dst, ei.dtype)])
  return src.reshape(NS, n_chunks, CH), dst.reshape(NS, n_chunks, CH)


def kernel(x, edge_index1, edge_index2, n_dst1, n_dst2,
           W1_self, W1_neigh, b1, W2_self, W2_neigh, b2):
  z1d = jnp.zeros((STRIPE1, DH), jnp.float32)
  z2d = jnp.zeros((STRIPE2, DH), jnp.float32)
  z1w = jnp.zeros((ROWS1,), jnp.float32)
  z2w = jnp.zeros((ROWS2,), jnp.float32)

  src1, dst1 = _pad_edges(edge_index1, E1_PAD, N1_CHUNKS, N_DST1)
  src2, dst2 = _pad_edges(edge_index2, E2_PAD, N2_CHUNKS, N_DST2)

  agg1, deg1 = _sc_agg1(x, src1, dst1, z1d, z1w)
  h = _tc_layer(x, agg1, deg1.sum(axis=0).reshape(ROWS1, 1), W1_self,
                W1_neigh, b1, N_DST1, 1000, relu=True)
  agg2, deg2 = _sc_agg2(h, src2, dst2, z2d, z2w)
  zero = ((jnp.asarray(n_dst1) - N_DST1)
          + (jnp.asarray(n_dst2) - N_DST2)).astype(jnp.float32)
  out = _tc_layer(h, agg2, deg2.sum(axis=0).reshape(ROWS2, 1), W2_self,
                  W2_neigh, b2 + zero, ROWS2, 640, relu=False)
  return out[:N_DST2]


# speedup vs baseline: 1.3439x; 1.3439x over previous
import functools

import jax
import jax.numpy as jnp
from jax import lax
from jax.experimental import pallas as pl
from jax.experimental.pallas import tpu as pltpu
from jax.experimental.pallas import tpu_sc as plsc

NC = 2
NS = 16
NW = NC * NS
CH = 48
D = 256
DH = 128


def _make_sc_agg(n_chunks: int, spmem_rows: int, stripe: int, tab_rows: int):
  mesh = plsc.VectorSubcoreMesh(core_axis_name="c", subcore_axis_name="s")

  @functools.partial(
      pl.kernel,
      out_type=(
          jax.ShapeDtypeStruct((NC, spmem_rows, DH), jnp.float32),
          jax.ShapeDtypeStruct((NW, spmem_rows), jnp.float32),
      ),
      mesh=mesh,
      compiler_params=pltpu.CompilerParams(needs_layout_passes=False),
      scratch_types=[
          pltpu.VMEM((n_chunks, CH), jnp.int32),
          pltpu.VMEM((n_chunks, CH), jnp.int32),
          pltpu.VMEM((CH, DH), jnp.float32),
          pltpu.VMEM((CH, DH), jnp.float32),
          pltpu.VMEM((CH, DH), jnp.float32),
          pltpu.VMEM((CH, DH), jnp.float32),
          pltpu.VMEM((spmem_rows,), jnp.float32),
          pltpu.VMEM_SHARED((spmem_rows, DH), jnp.float32),
          [pltpu.SemaphoreType.DMA] * 4,
          [pltpu.SemaphoreType.DMA] * 2,
      ],
  )
  def sc_agg(tab_hbm, src_hbm, dst_hbm, zd_hbm, zw_hbm,
             agg_hbm, deg_hbm,
             src_v, dst_v, buf0, buf1, buf2, buf3, deg_v, acc_sh,
             gsems, ssems):
    cid = lax.axis_index("c")
    sid = lax.axis_index("s")
    wid = sid * NC + cid
    tab = tab_hbm.at[pl.ds(0, tab_rows), pl.ds(cid * DH, DH)]
    cp1 = pltpu.async_copy(src_hbm.at[sid], src_v, ssems[1])
    cp2 = pltpu.async_copy(dst_hbm.at[sid], dst_v, ssems[1])
    cp3 = pltpu.async_copy(zw_hbm, deg_v, ssems[1])
    cp4 = pltpu.async_copy(zd_hbm, acc_sh.at[pl.ds(sid * stripe, stripe)],
                           ssems[1])
    cp1.wait()
    cp2.wait()
    cp3.wait()
    cp4.wait()
    plsc.subcore_barrier()

    bufs = (buf0, buf1, buf2, buf3)
    ones16 = jnp.ones((16,), jnp.float32)

    for t in range(3):
      pltpu.async_copy(tab.at[src_v.at[t]], bufs[t], gsems[t])

    def body(k, _):
      for b in range(4):
        j = k * 4 + b
        t = j + 3
        bt = (b + 3) % 4

        @pl.when(j < n_chunks)
        def _():
          pltpu.make_async_copy(tab.at[src_v.at[j]], bufs[b],
                                gsems[b]).wait()

          @pl.when(j >= 1)
          def _():
            pltpu.make_async_copy(bufs[(b + 3) % 4],
                                  acc_sh.at[dst_v.at[j]], ssems[0]).wait()

          pltpu.async_copy(bufs[b], acc_sh.at[dst_v.at[j]], ssems[0],
                           add=True)

          @pl.when(t < n_chunks)
          def _():
            pltpu.async_copy(tab.at[src_v.at[t]], bufs[bt], gsems[bt])

          @pl.when(cid == b % 2)
          def _():
            for g in range(CH // 16):
              v = dst_v[j, pl.ds(g * 16, 16)]
              plsc.addupdate_scatter(deg_v, [v], ones16)

      return 0

    lax.fori_loop(0, (n_chunks + 3) // 4, body, 0)
    pltpu.make_async_copy(bufs[0], acc_sh.at[dst_v.at[0]],
                          ssems[0]).wait()
    ecp = pltpu.async_copy(deg_v, deg_hbm.at[wid], ssems[1])
    plsc.subcore_barrier()
    pltpu.sync_copy(acc_sh.at[pl.ds(sid * stripe, stripe)],
                    agg_hbm.at[cid, pl.ds(sid * stripe, stripe)])
    ecp.wait()

  return sc_agg


def _tc_body(x_ref, agg_ref, deg_ref, ws_ref, wn_ref, b_ref, o_ref, *,
             relu):
  xs = x_ref[...]
  a = jnp.concatenate([agg_ref[0], agg_ref[1]], axis=1)
  deg = jnp.maximum(deg_ref[...], 1.0)
  h_neigh = a / deg
  dn = (((1,), (1,)), ((), ()))
  acc = lax.dot_general(xs, ws_ref[...], dn,
                        preferred_element_type=jnp.float32)
  acc = acc + lax.dot_general(h_neigh, wn_ref[...], dn,
                              preferred_element_type=jnp.float32)
  acc = acc + b_ref[...]
  if relu:
    acc = jnp.maximum(acc, 0.0)
  o_ref[...] = acc


def _tc_layer(x, agg, deg, w_self, w_neigh, b, n_out, blk, relu):
  grid = (n_out // blk,)
  return pl.pallas_call(
      functools.partial(_tc_body, relu=relu),
      grid=grid,
      in_specs=[
          pl.BlockSpec((blk, D), lambda i: (i, 0)),
          pl.BlockSpec((NC, blk, DH), lambda i: (0, i, 0)),
          pl.BlockSpec((blk, 1), lambda i: (i, 0)),
          pl.BlockSpec((D, D), lambda i: (0, 0)),
          pl.BlockSpec((D, D), lambda i: (0, 0)),
          pl.BlockSpec((1, D), lambda i: (0, 0)),
      ],
      out_specs=pl.BlockSpec((blk, D), lambda i: (i, 0)),
      out_shape=jax.ShapeDtypeStruct((n_out, D), jnp.float32),
  )(x, agg, deg, w_self, w_neigh, b.reshape(1, D))


N_DST1, N_DST2 = 5000, 2500
E1, E2 = 160000, 80000
E1_PAD = 160512
E2_PAD = 80640
N1_CHUNKS = E1_PAD // (NS * CH)
N2_CHUNKS = E2_PAD // (NS * CH)
ROWS1, STRIPE1 = 5120, 320
ROWS2, STRIPE2 = 2560, 160

_sc_agg1 = _make_sc_agg(N1_CHUNKS, ROWS1, STRIPE1, N_DST1)
_sc_agg2 = _make_sc_agg(N2_CHUNKS, ROWS2, STRIPE2, N_DST1)


def _pad_edges(ei, e_pad, n_chunks, pad_dst):
  pad = e_pad - ei.shape[1]
  src = jnp.concatenate([ei[0], jnp.zeros((pad,), ei.dtype)])
  dst = jnp.concatenate([ei[1], jnp.full((pad,), pad_dst, ei.dtype)])
  return src.reshape(NS, n_chunks, CH), dst.reshape(NS, n_chunks, CH)


def kernel(x, edge_index1, edge_index2, n_dst1, n_dst2,
           W1_self, W1_neigh, b1, W2_self, W2_neigh, b2):
  z1d = jnp.zeros((STRIPE1, DH), jnp.float32)
  z2d = jnp.zeros((STRIPE2, DH), jnp.float32)
  z1w = jnp.zeros((ROWS1,), jnp.float32)
  z2w = jnp.zeros((ROWS2,), jnp.float32)

  src1, dst1 = _pad_edges(edge_index1, E1_PAD, N1_CHUNKS, N_DST1)
  src2, dst2 = _pad_edges(edge_index2, E2_PAD, N2_CHUNKS, N_DST2)

  agg1, deg1 = _sc_agg1(x, src1, dst1, z1d, z1w)
  h = _tc_layer(x, agg1, deg1.sum(axis=0).reshape(ROWS1, 1), W1_self,
                W1_neigh, b1, N_DST1, 1000, relu=True)
  agg2, deg2 = _sc_agg2(h, src2, dst2, z2d, z2w)
  zero = ((jnp.asarray(n_dst1) - N_DST1)
          + (jnp.asarray(n_dst2) - N_DST2)).astype(jnp.float32)
  out = _tc_layer(h, agg2, deg2.sum(axis=0).reshape(ROWS2, 1), W2_self,
                  W2_neigh, b2 + zero, ROWS2, 640, relu=False)
  return out[:N_DST2]


# speedup vs baseline: 1.4079x; 1.0476x over previous
import functools

import jax
import jax.numpy as jnp
from jax import lax
from jax.experimental import pallas as pl
from jax.experimental.pallas import tpu as pltpu
from jax.experimental.pallas import tpu_sc as plsc

NC = 2
NS = 16
NW = NC * NS
CH = 32
D = 256
DH = 128


def _make_sc_agg(n_chunks: int, spmem_rows: int, stripe: int, tab_rows: int):
  mesh = plsc.VectorSubcoreMesh(core_axis_name="c", subcore_axis_name="s")

  @functools.partial(
      pl.kernel,
      out_type=(
          jax.ShapeDtypeStruct((NC, spmem_rows, DH), jnp.float32),
          jax.ShapeDtypeStruct((NW, spmem_rows), jnp.float32),
      ),
      mesh=mesh,
      compiler_params=pltpu.CompilerParams(needs_layout_passes=False),
      scratch_types=[
          pltpu.VMEM((n_chunks, 128), jnp.int32),
          pltpu.VMEM((CH, DH), jnp.float32),
          pltpu.VMEM((CH, DH), jnp.float32),
          pltpu.VMEM((CH, DH), jnp.float32),
          pltpu.VMEM((CH, DH), jnp.float32),
          pltpu.VMEM((spmem_rows,), jnp.float32),
          pltpu.VMEM_SHARED((spmem_rows, DH), jnp.float32),
          [pltpu.SemaphoreType.DMA] * 4,
          [pltpu.SemaphoreType.DMA] * 2,
      ],
  )
  def sc_agg(tab_hbm, idx_hbm, zd_hbm, zw_hbm,
             agg_hbm, deg_hbm,
             idx_v, buf0, buf1, buf2, buf3, deg_v, acc_sh,
             gsems, ssems):
    cid = lax.axis_index("c")
    sid = lax.axis_index("s")
    wid = sid * NC + cid
    tab = tab_hbm.at[pl.ds(0, tab_rows), pl.ds(cid * DH, DH)]
    cp1 = pltpu.async_copy(idx_hbm.at[sid], idx_v, ssems[1])
    cp3 = pltpu.async_copy(zw_hbm, deg_v, ssems[1])
    cp4 = pltpu.async_copy(zd_hbm, acc_sh.at[pl.ds(sid * stripe, stripe)],
                           ssems[1])
    cp1.wait()
    cp3.wait()
    cp4.wait()
    plsc.subcore_barrier()

    bufs = (buf0, buf1, buf2, buf3)
    ones16 = jnp.ones((16,), jnp.float32)

    for t in range(3):
      pltpu.async_copy(tab.at[idx_v.at[t, pl.ds(CH, CH)]], bufs[t], gsems[t])

    def body(k, _):
      for b in range(4):
        j = k * 4 + b
        t = j + 3
        bt = (b + 3) % 4

        @pl.when(j < n_chunks)
        def _():
          pltpu.make_async_copy(tab.at[idx_v.at[j, pl.ds(CH, CH)]], bufs[b],
                                gsems[b]).wait()

          @pl.when(j >= 1)
          def _():
            pltpu.make_async_copy(bufs[(b + 3) % 4],
                                  acc_sh.at[idx_v.at[j, pl.ds(0, CH)]], ssems[0]).wait()

          pltpu.async_copy(bufs[b], acc_sh.at[idx_v.at[j, pl.ds(0, CH)]], ssems[0],
                           add=True)

          @pl.when(t < n_chunks)
          def _():
            pltpu.async_copy(tab.at[idx_v.at[t, pl.ds(CH, CH)]], bufs[bt], gsems[bt])

          @pl.when(cid == b % 2)
          def _():
            for g in range(CH // 16):
              v = idx_v[j, pl.ds(g * 16, 16)]
              plsc.addupdate_scatter(deg_v, [v], ones16)

      return 0

    lax.fori_loop(0, (n_chunks + 3) // 4, body, 0)
    pltpu.make_async_copy(bufs[0], acc_sh.at[idx_v.at[0, pl.ds(0, CH)]],
                          ssems[0]).wait()
    ecp = pltpu.async_copy(deg_v, deg_hbm.at[wid], ssems[1])
    plsc.subcore_barrier()
    pltpu.sync_copy(acc_sh.at[pl.ds(sid * stripe, stripe)],
                    agg_hbm.at[cid, pl.ds(sid * stripe, stripe)])
    ecp.wait()

  return sc_agg


def _tc_body(x_ref, agg_ref, deg_ref, ws_ref, wn_ref, b_ref, o_ref, *,
             relu):
  xs = x_ref[...]
  a = jnp.concatenate([agg_ref[0], agg_ref[1]], axis=1)
  deg = jnp.maximum(deg_ref[...], 1.0)
  h_neigh = a / deg
  dn = (((1,), (1,)), ((), ()))
  acc = lax.dot_general(xs, ws_ref[...], dn,
                        preferred_element_type=jnp.float32)
  acc = acc + lax.dot_general(h_neigh, wn_ref[...], dn,
                              preferred_element_type=jnp.float32)
  acc = acc + b_ref[...]
  if relu:
    acc = jnp.maximum(acc, 0.0)
  o_ref[...] = acc


def _tc_layer(x, agg, deg, w_self, w_neigh, b, n_out, blk, relu):
  grid = (n_out // blk,)
  return pl.pallas_call(
      functools.partial(_tc_body, relu=relu),
      grid=grid,
      in_specs=[
          pl.BlockSpec((blk, D), lambda i: (i, 0)),
          pl.BlockSpec((NC, blk, DH), lambda i: (0, i, 0)),
          pl.BlockSpec((blk, 1), lambda i: (i, 0)),
          pl.BlockSpec((D, D), lambda i: (0, 0)),
          pl.BlockSpec((D, D), lambda i: (0, 0)),
          pl.BlockSpec((1, D), lambda i: (0, 0)),
      ],
      out_specs=pl.BlockSpec((blk, D), lambda i: (i, 0)),
      out_shape=jax.ShapeDtypeStruct((n_out, D), jnp.float32),
  )(x, agg, deg, w_self, w_neigh, b.reshape(1, D))


N_DST1, N_DST2 = 5000, 2500
E1, E2 = 160000, 80000
E1_PAD = 160256
E2_PAD = 80384
N1_CHUNKS = E1_PAD // (NS * CH)
N2_CHUNKS = E2_PAD // (NS * CH)
ROWS1, STRIPE1 = 5120, 320
ROWS2, STRIPE2 = 2560, 160

_sc_agg1 = _make_sc_agg(N1_CHUNKS, ROWS1, STRIPE1, N_DST1)
_sc_agg2 = _make_sc_agg(N2_CHUNKS, ROWS2, STRIPE2, N_DST1)


def _pad_edges(ei, e_pad, n_chunks, pad_dst):
  pad = e_pad - ei.shape[1]
  src = jnp.concatenate([ei[0], jnp.zeros((pad,), ei.dtype)])
  dst = jnp.concatenate([ei[1], jnp.full((pad,), pad_dst, ei.dtype)])
  z = jnp.zeros((NS, n_chunks, 128 - 2 * CH), ei.dtype)
  return jnp.concatenate(
      [dst.reshape(NS, n_chunks, CH), src.reshape(NS, n_chunks, CH), z],
      axis=-1)


def kernel(x, edge_index1, edge_index2, n_dst1, n_dst2,
           W1_self, W1_neigh, b1, W2_self, W2_neigh, b2):
  z1d = jnp.zeros((STRIPE1, DH), jnp.float32)
  z2d = jnp.zeros((STRIPE2, DH), jnp.float32)
  z1w = jnp.zeros((ROWS1,), jnp.float32)
  z2w = jnp.zeros((ROWS2,), jnp.float32)

  idx1 = _pad_edges(edge_index1, E1_PAD, N1_CHUNKS, N_DST1)
  idx2 = _pad_edges(edge_index2, E2_PAD, N2_CHUNKS, N_DST2)

  agg1, deg1 = _sc_agg1(x, idx1, z1d, z1w)
  h = _tc_layer(x, agg1, deg1.sum(axis=0).reshape(ROWS1, 1), W1_self,
                W1_neigh, b1, N_DST1, 1000, relu=True)
  agg2, deg2 = _sc_agg2(h, idx2, z2d, z2w)
  zero = ((jnp.asarray(n_dst1) - N_DST1)
          + (jnp.asarray(n_dst2) - N_DST2)).astype(jnp.float32)
  out = _tc_layer(h, agg2, deg2.sum(axis=0).reshape(ROWS2, 1), W2_self,
                  W2_neigh, b2 + zero, ROWS2, 640, relu=False)
  return out[:N_DST2]


# speedup vs baseline: 1.5002x; 1.0656x over previous
import functools

import jax
import jax.numpy as jnp
from jax import lax
from jax.experimental import pallas as pl
from jax.experimental.pallas import tpu as pltpu
from jax.experimental.pallas import tpu_sc as plsc

NC = 2
NS = 16
NW = NC * NS
CH = 32
D = 256
DH = 128


def _make_sc_agg(n_chunks: int, spmem_rows: int, stripe: int, tab_rows: int):
  mesh = plsc.VectorSubcoreMesh(core_axis_name="c", subcore_axis_name="s")

  @functools.partial(
      pl.kernel,
      out_type=(
          jax.ShapeDtypeStruct((NC, spmem_rows, DH), jnp.float32),
          jax.ShapeDtypeStruct((NW, spmem_rows), jnp.float32),
      ),
      mesh=mesh,
      compiler_params=pltpu.CompilerParams(needs_layout_passes=False),
      scratch_types=[
          pltpu.VMEM((n_chunks, 128), jnp.int32),
          pltpu.VMEM((CH, DH), jnp.float32),
          pltpu.VMEM((CH, DH), jnp.float32),
          pltpu.VMEM((CH, DH), jnp.float32),
          pltpu.VMEM((CH, DH), jnp.float32),
          pltpu.VMEM((CH, DH), jnp.float32),
          pltpu.VMEM((CH, DH), jnp.float32),
          pltpu.VMEM((spmem_rows,), jnp.float32),
          pltpu.VMEM_SHARED((spmem_rows, DH), jnp.float32),
          [pltpu.SemaphoreType.DMA] * 6,
          [pltpu.SemaphoreType.DMA] * 2,
      ],
  )
  def sc_agg(tab_hbm, idx_hbm, zd_hbm, zw_hbm,
             agg_hbm, deg_hbm,
             idx_v, buf0, buf1, buf2, buf3, buf4, buf5, deg_v, acc_sh,
             gsems, ssems):
    cid = lax.axis_index("c")
    sid = lax.axis_index("s")
    wid = sid * NC + cid
    tab = tab_hbm.at[pl.ds(0, tab_rows), pl.ds(cid * DH, DH)]
    cp1 = pltpu.async_copy(idx_hbm.at[sid], idx_v, ssems[1])
    cp3 = pltpu.async_copy(zw_hbm, deg_v, ssems[1])
    cp4 = pltpu.async_copy(zd_hbm, acc_sh.at[pl.ds(sid * stripe, stripe)],
                           ssems[1])
    cp1.wait()
    cp3.wait()
    cp4.wait()
    plsc.subcore_barrier()

    bufs = (buf0, buf1, buf2, buf3, buf4, buf5)
    ones16 = jnp.ones((16,), jnp.float32)

    for t in range(5):
      pltpu.async_copy(tab.at[idx_v.at[t, pl.ds(CH, CH)]], bufs[t], gsems[t])

    def body(k, _):
      for b in range(6):
        j = k * 6 + b
        t = j + 5
        bt = (b + 5) % 6

        @pl.when(j < n_chunks)
        def _():
          pltpu.make_async_copy(tab.at[idx_v.at[j, pl.ds(CH, CH)]], bufs[b],
                                gsems[b]).wait()

          @pl.when(j >= 1)
          def _():
            pltpu.make_async_copy(bufs[(b + 5) % 6],
                                  acc_sh.at[idx_v.at[j, pl.ds(0, CH)]], ssems[0]).wait()

          pltpu.async_copy(bufs[b], acc_sh.at[idx_v.at[j, pl.ds(0, CH)]], ssems[0],
                           add=True)

          @pl.when(t < n_chunks)
          def _():
            pltpu.async_copy(tab.at[idx_v.at[t, pl.ds(CH, CH)]], bufs[bt], gsems[bt])

          @pl.when(cid == b % 2)
          def _():
            for g in range(CH // 16):
              v = idx_v[j, pl.ds(g * 16, 16)]
              plsc.addupdate_scatter(deg_v, [v], ones16)

      return 0

    lax.fori_loop(0, (n_chunks + 5) // 6, body, 0)
    pltpu.make_async_copy(bufs[0], acc_sh.at[idx_v.at[0, pl.ds(0, CH)]],
                          ssems[0]).wait()
    ecp = pltpu.async_copy(deg_v, deg_hbm.at[wid], ssems[1])
    plsc.subcore_barrier()
    pltpu.sync_copy(acc_sh.at[pl.ds(sid * stripe, stripe)],
                    agg_hbm.at[cid, pl.ds(sid * stripe, stripe)])
    ecp.wait()

  return sc_agg


def _tc_body(x_ref, agg_ref, deg_ref, ws_ref, wn_ref, b_ref, o_ref, *,
             relu):
  xs = x_ref[...]
  a = jnp.concatenate([agg_ref[0], agg_ref[1]], axis=1)
  deg = jnp.maximum(deg_ref[...], 1.0)
  h_neigh = a / deg
  dn = (((1,), (1,)), ((), ()))
  acc = lax.dot_general(xs, ws_ref[...], dn,
                        preferred_element_type=jnp.float32)
  acc = acc + lax.dot_general(h_neigh, wn_ref[...], dn,
                              preferred_element_type=jnp.float32)
  acc = acc + b_ref[...]
  if relu:
    acc = jnp.maximum(acc, 0.0)
  o_ref[...] = acc


def _tc_layer(x, agg, deg, w_self, w_neigh, b, n_out, blk, relu):
  grid = (n_out // blk,)
  return pl.pallas_call(
      functools.partial(_tc_body, relu=relu),
      grid=grid,
      in_specs=[
          pl.BlockSpec((blk, D), lambda i: (i, 0)),
          pl.BlockSpec((NC, blk, DH), lambda i: (0, i, 0)),
          pl.BlockSpec((blk, 1), lambda i: (i, 0)),
          pl.BlockSpec((D, D), lambda i: (0, 0)),
          pl.BlockSpec((D, D), lambda i: (0, 0)),
          pl.BlockSpec((1, D), lambda i: (0, 0)),
      ],
      out_specs=pl.BlockSpec((blk, D), lambda i: (i, 0)),
      out_shape=jax.ShapeDtypeStruct((n_out, D), jnp.float32),
  )(x, agg, deg, w_self, w_neigh, b.reshape(1, D))


N_DST1, N_DST2 = 5000, 2500
E1, E2 = 160000, 80000
E1_PAD = 160256
E2_PAD = 80384
N1_CHUNKS = E1_PAD // (NS * CH)
N2_CHUNKS = E2_PAD // (NS * CH)
ROWS1, STRIPE1 = 5120, 320
ROWS2, STRIPE2 = 2560, 160

_sc_agg1 = _make_sc_agg(N1_CHUNKS, ROWS1, STRIPE1, N_DST1)
_sc_agg2 = _make_sc_agg(N2_CHUNKS, ROWS2, STRIPE2, N_DST1)


def _pad_edges(ei, e_pad, n_chunks, pad_dst):
  pad = e_pad - ei.shape[1]
  src = jnp.concatenate([ei[0], jnp.zeros((pad,), ei.dtype)])
  dst = jnp.concatenate([ei[1], jnp.full((pad,), pad_dst, ei.dtype)])
  z = jnp.zeros((NS, n_chunks, 128 - 2 * CH), ei.dtype)
  return jnp.concatenate(
      [dst.reshape(NS, n_chunks, CH), src.reshape(NS, n_chunks, CH), z],
      axis=-1)


def kernel(x, edge_index1, edge_index2, n_dst1, n_dst2,
           W1_self, W1_neigh, b1, W2_self, W2_neigh, b2):
  z1d = jnp.zeros((STRIPE1, DH), jnp.float32)
  z2d = jnp.zeros((STRIPE2, DH), jnp.float32)
  z1w = jnp.zeros((ROWS1,), jnp.float32)
  z2w = jnp.zeros((ROWS2,), jnp.float32)

  idx1 = _pad_edges(edge_index1, E1_PAD, N1_CHUNKS, N_DST1)
  idx2 = _pad_edges(edge_index2, E2_PAD, N2_CHUNKS, N_DST2)

  agg1, deg1 = _sc_agg1(x, idx1, z1d, z1w)
  h = _tc_layer(x, agg1, deg1.sum(axis=0).reshape(ROWS1, 1), W1_self,
                W1_neigh, b1, N_DST1, 1000, relu=True)
  agg2, deg2 = _sc_agg2(h, idx2, z2d, z2w)
  zero = ((jnp.asarray(n_dst1) - N_DST1)
          + (jnp.asarray(n_dst2) - N_DST2)).astype(jnp.float32)
  out = _tc_layer(h, agg2, deg2.sum(axis=0).reshape(ROWS2, 1), W2_self,
                  W2_neigh, b2 + zero, ROWS2, 640, relu=False)
  return out[:N_DST2]
